# Initial kernel scaffold; baseline (speedup 1.0000x reference)
import jax
import jax.numpy as jnp
from jax.experimental import pallas as pl


def kernel(X, kmer, emb_table):
    raise NotImplementedError("write your pallas kernel here")


# speedup vs baseline: 4.3844x; 4.3844x over previous
import functools

import jax
import jax.numpy as jnp
from jax import lax
from jax.experimental import pallas as pl
from jax.experimental.pallas import tpu as pltpu
from jax.experimental.pallas import tpu_sc as plsc

_NC = 2
_NS = 16
_NW = _NC * _NS
_N = 16384
_K = 3
_D = 16
_B = _N * _K
_BPW = _B // _NW
_CHUNK = 128
_NCHUNK = _BPW // _CHUNK


def _build_gather():
    mesh = plsc.VectorSubcoreMesh(core_axis_name="c", subcore_axis_name="s")

    @functools.partial(
        pl.kernel,
        mesh=mesh,
        out_type=jax.ShapeDtypeStruct((_B, _D), jnp.float32),
        scratch_types=[
            pltpu.VMEM((_NCHUNK, _CHUNK), jnp.int32),
            pltpu.VMEM((_BPW, _D), jnp.float32),
            pltpu.SemaphoreType.DMA,
        ],
        compiler_params=pltpu.CompilerParams(use_tc_tiling_on_sc=False),
    )
    def gather_kernel(idx_hbm, table_hbm, out_hbm, idx_v, rows_v, sem):
        wid = lax.axis_index("s") * _NC + lax.axis_index("c")
        pltpu.sync_copy(idx_hbm.at[wid], idx_v)
        copies = []
        for j in range(_NCHUNK):
            copies.append(
                pltpu.async_copy(
                    table_hbm.at[idx_v.at[j]],
                    rows_v.at[pl.ds(j * _CHUNK, _CHUNK)],
                    sem,
                )
            )
        for c in copies:
            c.wait()
        pltpu.sync_copy(rows_v, out_hbm.at[pl.ds(wid * _BPW, _BPW)])

    return gather_kernel


_gather = _build_gather()


def kernel(X, kmer, emb_table):
    idx = kmer.astype(jnp.int32).reshape(_NW, _NCHUNK, _CHUNK)
    rows = _gather(idx, emb_table)
    return (X, rows.reshape(_N, _K * _D))


# speedup vs baseline: 5.4103x; 1.2340x over previous
import functools

import jax
import jax.numpy as jnp
from jax import lax
from jax.experimental import pallas as pl
from jax.experimental.pallas import tpu as pltpu
from jax.experimental.pallas import tpu_sc as plsc

_NC = 2
_NS = 16
_NW = _NC * _NS
_N = 16384
_K = 3
_D = 16
_J = _K * _D
_V = 1024
_NPW = _N // _NW
_L = 16
_GRP = _NPW // _L


def _build_gather():
    mesh = plsc.VectorSubcoreMesh(core_axis_name="c", subcore_axis_name="s")

    @functools.partial(
        pl.kernel,
        mesh=mesh,
        out_type=jax.ShapeDtypeStruct((_J, _N), jnp.float32),
        scratch_types=[
            pltpu.VMEM((_K * _NPW,), jnp.int32),
            pltpu.VMEM((_V * _D,), jnp.float32),
            pltpu.VMEM((_J * _NPW,), jnp.float32),
            pltpu.SemaphoreType.DMA,
        ],
        compiler_params=pltpu.CompilerParams(needs_layout_passes=False),
    )
    def gather_kernel(kmer_t_hbm, table_hbm, out_hbm, idx_v, tab_v, m_v, sem):
        wid = lax.axis_index("s") * _NC + lax.axis_index("c")
        n0 = wid * _NPW
        pltpu.sync_copy(table_hbm, tab_v)
        for k in range(_K):
            pltpu.sync_copy(
                kmer_t_hbm.at[pl.ds(k * _N + n0, _NPW)],
                idx_v.at[pl.ds(k * _NPW, _NPW)],
            )

        def body(i, carry):
            for k in range(_K):
                v = idx_v[pl.ds(k * _NPW + i * _L, _L)]
                voff = v * _D
                for c in range(_D):
                    vals = plsc.load_gather(tab_v, [voff + c])
                    j = k * _D + c
                    m_v[pl.ds(j * _NPW + i * _L, _L)] = vals
            return carry

        lax.fori_loop(0, _GRP, body, 0)

        copies = []
        for j in range(_J):
            copies.append(
                pltpu.async_copy(
                    m_v.at[pl.ds(j * _NPW, _NPW)],
                    out_hbm.at[j, pl.ds(n0, _NPW)],
                    sem,
                )
            )
        for c in copies:
            c.wait()

    return gather_kernel


_gather = _build_gather()


def kernel(X, kmer, emb_table):
    kmer_t = kmer.astype(jnp.int32).T.reshape(-1)
    m = _gather(kmer_t, emb_table.reshape(-1))
    return (X, m.T)


# speedup vs baseline: 6.0494x; 1.1181x over previous
import functools

import jax
import jax.numpy as jnp
from jax import lax
from jax.experimental import pallas as pl
from jax.experimental.pallas import tpu as pltpu
from jax.experimental.pallas import tpu_sc as plsc

_NC = 2
_NS = 16
_NW = _NC * _NS
_N = 16384
_K = 3
_D = 16
_J = _K * _D
_V = 1024
_NPW = _N // _NW
_L = 16
_GRP = _NPW // _L


def _build_gather():
    mesh = plsc.VectorSubcoreMesh(core_axis_name="c", subcore_axis_name="s")

    @functools.partial(
        pl.kernel,
        mesh=mesh,
        out_type=jax.ShapeDtypeStruct((_J, _N), jnp.float32),
        scratch_types=[
            pltpu.VMEM((_K * _NPW,), jnp.int32),
            pltpu.VMEM((_V * _D,), jnp.float32),
            pltpu.VMEM((_J * _NPW,), jnp.float32),
            pltpu.SemaphoreType.DMA,
        ],
        compiler_params=pltpu.CompilerParams(needs_layout_passes=False),
    )
    def gather_kernel(kmer_t_hbm, table_hbm, out_hbm, idx_v, tab_v, m_v, sem):
        wid = lax.axis_index("s") * _NC + lax.axis_index("c")
        n0 = wid * _NPW
        pltpu.sync_copy(table_hbm, tab_v)
        for k in range(_K):
            pltpu.sync_copy(
                kmer_t_hbm.at[pl.ds(k * _N + n0, _NPW)],
                idx_v.at[pl.ds(k * _NPW, _NPW)],
            )

        @plsc.parallel_loop(0, _GRP, unroll=2)
        def _body(i):
            base = i * _L
            voffs = [
                idx_v[pl.ds(k * _NPW + base, _L)] * _D for k in range(_K)
            ]
            offs = [voffs[k] + c for k in range(_K) for c in range(_D)]
            vals = [plsc.load_gather(tab_v, [o]) for o in offs]
            for j in range(_J):
                m_v[pl.ds(j * _NPW + base, _L)] = vals[j]

        copies = []
        for j in range(_J):
            copies.append(
                pltpu.async_copy(
                    m_v.at[pl.ds(j * _NPW, _NPW)],
                    out_hbm.at[j, pl.ds(n0, _NPW)],
                    sem,
                )
            )
        for c in copies:
            c.wait()

    return gather_kernel


_gather = _build_gather()


def kernel(X, kmer, emb_table):
    kmer_t = kmer.astype(jnp.int32).T.reshape(-1)
    m = _gather(kmer_t, emb_table.reshape(-1))
    return (X, m.T)


# speedup vs baseline: 6.5429x; 1.0816x over previous
import functools

import jax
import jax.numpy as jnp
from jax import lax
from jax.experimental import pallas as pl
from jax.experimental.pallas import tpu as pltpu
from jax.experimental.pallas import tpu_sc as plsc

_NC = 2
_NS = 16
_NW = _NC * _NS
_N = 16384
_K = 3
_D = 16
_J = _K * _D
_V = 1024
_NPW = _N // _NW
_L = 16
_GRP = _NPW // _L


def _build_gather():
    mesh = plsc.VectorSubcoreMesh(core_axis_name="c", subcore_axis_name="s")

    @functools.partial(
        pl.kernel,
        mesh=mesh,
        out_type=jax.ShapeDtypeStruct((_J, _N), jnp.float32),
        scratch_types=[
            pltpu.VMEM((_K * _NPW,), jnp.int32),
            pltpu.VMEM((_V * _D,), jnp.float32),
            pltpu.VMEM((_J * _NPW,), jnp.float32),
            pltpu.SemaphoreType.DMA,
        ],
        compiler_params=pltpu.CompilerParams(needs_layout_passes=False),
    )
    def gather_kernel(kmer_t_hbm, table_hbm, out_hbm, idx_v, tab_v, m_v, sem):
        wid = lax.axis_index("s") * _NC + lax.axis_index("c")
        n0 = wid * _NPW
        pltpu.sync_copy(table_hbm, tab_v)
        for k in range(_K):
            pltpu.sync_copy(
                kmer_t_hbm.at[pl.ds(k * _N + n0, _NPW)],
                idx_v.at[pl.ds(k * _NPW, _NPW)],
            )

        @plsc.parallel_loop(0, _GRP, unroll=4)
        def _body(i):
            base = i * _L
            voffs = [
                idx_v[pl.ds(k * _NPW + base, _L)] * _D for k in range(_K)
            ]
            offs = [voffs[k] + c for k in range(_K) for c in range(_D)]
            vals = [plsc.load_gather(tab_v, [o]) for o in offs]
            for j in range(_J):
                m_v[pl.ds(j * _NPW + base, _L)] = vals[j]

        copies = []
        for j in range(_J):
            copies.append(
                pltpu.async_copy(
                    m_v.at[pl.ds(j * _NPW, _NPW)],
                    out_hbm.at[j, pl.ds(n0, _NPW)],
                    sem,
                )
            )
        for c in copies:
            c.wait()

    return gather_kernel


_gather = _build_gather()


def kernel(X, kmer, emb_table):
    kmer_t = kmer.astype(jnp.int32).T.reshape(-1)
    m = _gather(kmer_t, emb_table.reshape(-1))
    return (X, m.T)
